# Initial kernel scaffold; baseline (speedup 1.0000x reference)
import jax
import jax.numpy as jnp
from jax.experimental import pallas as pl


def kernel(x, params):
    raise NotImplementedError("write your pallas kernel here")


# speedup vs baseline: 3.6552x; 3.6552x over previous
import functools

import jax
import jax.numpy as jnp
from jax import lax
from jax.experimental import pallas as pl
from jax.experimental.pallas import tpu as pltpu
from jax.experimental.pallas import tpu_sc as plsc

KNN = 16
NBR = KNN + 1
CIN = 6
HID = 128
LAT = 256

RB = 128
CW = 512
PB = 128


def _knn_mlp_body(xb_ref, ptc_ref, w1_ref, b1_ref, w2_ref, b2_ref,
                  idx_ref, h_ref):
    i = pl.program_id(1)
    n = xb_ref.shape[1] * pl.num_programs(1)
    nch = ptc_ref.shape[1]

    xb = xb_ref[0]
    h1 = jnp.maximum(
        jnp.dot(xb, w1_ref[...], preferred_element_type=jnp.float32)
        + b1_ref[...], 0.0)
    h_ref[0] = (jnp.dot(h1, w2_ref[...], preferred_element_type=jnp.float32)
                + b2_ref[...])

    lane8 = lax.broadcasted_iota(jnp.int32, (RB, 8), 1)
    xp = jnp.where(lane8 < 3, xb, 0.0)
    sqr = jnp.sum(xp * xp, axis=1)
    row_g = i * RB + lax.broadcasted_iota(jnp.int32, (RB, CW), 0)
    col_iota = lax.broadcasted_iota(jnp.int32, (RB, CW), 1)
    klane = lax.broadcasted_iota(jnp.int32, (RB, KNN), 1)
    lane2k = lax.broadcasted_iota(jnp.int32, (RB, 2 * KNN), 1)
    big = jnp.int32(2 ** 30)

    def chunk_step(c, carry):
        rd, ri = carry
        pt = ptc_ref[0, c]
        sqc = jnp.sum(pt * pt, axis=0)
        cross = jnp.dot(xp, pt, preferred_element_type=jnp.float32)
        d2 = sqr[:, None] + sqc[None, :] - 2.0 * cross
        col_g = c * CW + col_iota
        d2 = jnp.where(col_g == row_g, d2 + 1e10, d2)

        def extract(t, ec):
            d2c, cd, ci = ec
            m = jnp.min(d2c, axis=1)
            am = jnp.min(jnp.where(d2c <= m[:, None], col_g, big), axis=1)
            cd = jnp.where(klane == t, m[:, None], cd)
            ci = jnp.where(klane == t, am[:, None], ci)
            d2c = jnp.where(col_g == am[:, None], jnp.inf, d2c)
            return d2c, cd, ci

        cd0 = jnp.full((RB, KNN), jnp.inf, jnp.float32)
        ci0 = jnp.zeros((RB, KNN), jnp.int32)
        _, cd, ci = lax.fori_loop(0, KNN, extract, (d2, cd0, ci0))

        md = jnp.concatenate([rd, cd], axis=1)
        mi = jnp.concatenate([ri, ci], axis=1)

        def merge(t, mc):
            md, mi, nd, ni = mc
            m = jnp.min(md, axis=1)
            sel = md <= m[:, None]
            am = jnp.min(jnp.where(sel, mi, big), axis=1)
            nd = jnp.where(klane == t, m[:, None], nd)
            ni = jnp.where(klane == t, am[:, None], ni)
            md = jnp.where(sel & (mi == am[:, None]), jnp.inf, md)
            return md, mi, nd, ni

        _, _, nd, ni = lax.fori_loop(
            0, KNN, merge,
            (md, mi, jnp.zeros((RB, KNN), jnp.float32),
             jnp.zeros((RB, KNN), jnp.int32)))
        return nd, ni

    rd0 = jnp.full((RB, KNN), jnp.inf, jnp.float32)
    ri0 = jnp.zeros((RB, KNN), jnp.int32)
    _, ri = lax.fori_loop(0, nch, chunk_step, (rd0, ri0))
    idx_ref[0] = ri


def _knn_mlp(xb8, ptc, w1, b1, w2, b2):
    b, n, _ = xb8.shape
    nblk = n // RB
    grid = (b, nblk)
    return pl.pallas_call(
        _knn_mlp_body,
        grid=grid,
        in_specs=[
            pl.BlockSpec((1, RB, 8), lambda bi, i: (bi, i, 0)),
            pl.BlockSpec((1,) + ptc.shape[1:], lambda bi, i: (bi, 0, 0, 0)),
            pl.BlockSpec(w1.shape, lambda bi, i: (0, 0)),
            pl.BlockSpec(b1.shape, lambda bi, i: (0, 0)),
            pl.BlockSpec(w2.shape, lambda bi, i: (0, 0)),
            pl.BlockSpec(b2.shape, lambda bi, i: (0, 0)),
        ],
        out_specs=[
            pl.BlockSpec((1, RB, KNN), lambda bi, i: (bi, i, 0)),
            pl.BlockSpec((1, RB, HID), lambda bi, i: (bi, i, 0)),
        ],
        out_shape=[
            jax.ShapeDtypeStruct((b, n, KNN), jnp.int32),
            jax.ShapeDtypeStruct((b, n, HID), jnp.float32),
        ],
    )(xb8, ptc, w1, b1, w2, b2)


def _sc_gather(table, idx, chunk):
    info = plsc.get_sparse_core_info()
    nc, ns = info.num_cores, info.num_subcores
    nw = nc * ns
    e = idx.shape[0]
    d = table.shape[1]
    b_per_w = e // nw
    steps = b_per_w // chunk
    mesh = plsc.VectorSubcoreMesh(core_axis_name="c", subcore_axis_name="s")

    @functools.partial(
        pl.kernel, mesh=mesh,
        out_type=jax.ShapeDtypeStruct((e, d), jnp.float32),
        scratch_types=[
            pltpu.VMEM((chunk,), jnp.int32),
            pltpu.VMEM((chunk, d), jnp.float32),
            pltpu.SemaphoreType.DMA,
        ],
    )
    def gather_k(table_hbm, idx_hbm, out_hbm, idx_v, rows_v, sem):
        wid = lax.axis_index("s") * nc + lax.axis_index("c")
        base = wid * b_per_w

        def step(i, carry):
            off = base + i * chunk
            pltpu.sync_copy(idx_hbm.at[pl.ds(off, chunk)], idx_v)
            pltpu.async_copy(table_hbm.at[idx_v], rows_v, sem).wait()
            pltpu.sync_copy(rows_v, out_hbm.at[pl.ds(off, chunk)])
            return carry

        lax.fori_loop(0, steps, step, 0)

    return gather_k(table, idx)


def _pt_layer_body(g_ref, pg_ref, pi_ref, ws_ref, wv_ref, wp_ref, bp_ref,
                   out_ref, *, cout, reduce_max):
    g = g_ref[0]
    s = jnp.dot(g, ws_ref[...], preferred_element_type=jnp.float32)
    v = jnp.dot(g, wv_ref[...], preferred_element_type=jnp.float32)
    pg = pg_ref[0][:, :16]
    pi = pi_ref[0]
    rel = jnp.broadcast_to(pi[:, None, :], (PB, NBR, 16)).reshape(
        PB * NBR, 16) - pg
    delta = (jnp.dot(rel, wp_ref[...], preferred_element_type=jnp.float32)
             + bp_ref[...])
    a = (delta - s).reshape(PB, NBR, cout)
    m = jnp.max(a, axis=1, keepdims=True)
    ex = jnp.exp(a - m)
    z = jnp.sum(ex, axis=1)
    w = ex * (v + delta).reshape(PB, NBR, cout)
    o = jnp.sum(w, axis=1) / z
    if reduce_max:
        out_ref[0, 0, 0] = jnp.max(o, axis=0)
    else:
        out_ref[0] = o


def _pt_layer(gt, posg, pos16, ws, wv, wp, bp, cout, reduce_max):
    b, ne, _ = gt.shape
    n = ne // NBR
    nblk = n // PB
    body = functools.partial(_pt_layer_body, cout=cout, reduce_max=reduce_max)
    if reduce_max:
        out_spec = pl.BlockSpec((1, 1, 1, cout), lambda bi, i: (bi, i, 0, 0))
        out_shape = jax.ShapeDtypeStruct((b, nblk, 1, cout), jnp.float32)
    else:
        out_spec = pl.BlockSpec((1, PB, cout), lambda bi, i: (bi, i, 0))
        out_shape = jax.ShapeDtypeStruct((b, n, cout), jnp.float32)
    return pl.pallas_call(
        body,
        grid=(b, nblk),
        in_specs=[
            pl.BlockSpec((1, PB * NBR, HID), lambda bi, i: (bi, i, 0)),
            pl.BlockSpec((1, PB * NBR, HID), lambda bi, i: (bi, i, 0)),
            pl.BlockSpec((1, PB, 16), lambda bi, i: (bi, i, 0)),
            pl.BlockSpec(ws.shape, lambda bi, i: (0, 0)),
            pl.BlockSpec(wv.shape, lambda bi, i: (0, 0)),
            pl.BlockSpec(wp.shape, lambda bi, i: (0, 0)),
            pl.BlockSpec(bp.shape, lambda bi, i: (0, 0)),
        ],
        out_specs=out_spec,
        out_shape=out_shape,
    )(gt, posg, pos16, ws, wv, wp, bp)


def _pad_w(w, rows):
    return jnp.pad(w, ((0, rows - w.shape[0]), (0, 0)))


def kernel(x, params):
    b, cin, n = x.shape
    xb = jnp.transpose(x, (0, 2, 1))
    xb8 = jnp.pad(xb, ((0, 0), (0, 0), (0, 8 - cin)))
    post = jnp.pad(x[:, :3, :], ((0, 0), (0, 5), (0, 0)))
    nch = n // CW
    ptc = post.reshape(b, 8, nch, CW).transpose(0, 2, 1, 3)

    w1 = _pad_w(params['W1'], 8)
    b1 = params['b1'].reshape(1, HID)
    w2 = params['W2']
    b2 = params['b2'].reshape(1, HID)

    idx, h = _knn_mlp(xb8, ptc, w1, b1, w2, b2)

    self_col = jnp.broadcast_to(
        jnp.arange(n, dtype=jnp.int32)[None, :, None], (b, n, 1))
    nbr = jnp.concatenate([idx, self_col], axis=2)
    offs = (jnp.arange(b, dtype=jnp.int32) * n)[:, None, None]
    flat_idx = (nbr + offs).reshape(b * n * NBR)

    pos16 = jnp.pad(xb[:, :, :3], ((0, 0), (0, 0), (0, 13)))
    pos128 = jnp.pad(xb[:, :, :3], ((0, 0), (0, 0), (0, HID - 3)))
    posg = _sc_gather(pos128.reshape(b * n, HID), flat_idx, 272)
    posg = posg.reshape(b, n * NBR, HID)

    for name, cout, last in (('pt1', HID, False), ('pt2', HID, False),
                             ('pt3', LAT, True)):
        p = params[name]
        gt = _sc_gather(h.reshape(b * n, HID), flat_idx, 272)
        gt = gt.reshape(b, n * NBR, HID)
        h = _pt_layer(gt, posg, pos16, p['Ws'], p['Wv'],
                      _pad_w(p['Wp'], 16), p['bp'].reshape(1, cout),
                      cout, last)

    return jnp.max(h, axis=(1, 2))


# speedup vs baseline: 4.3364x; 1.1864x over previous
import functools

import jax
import jax.numpy as jnp
from jax import lax
from jax.experimental import pallas as pl
from jax.experimental.pallas import tpu as pltpu
from jax.experimental.pallas import tpu_sc as plsc

KNN = 16
NBR = KNN + 1
CIN = 6
HID = 128
LAT = 256

RB = 128
CW = 512
PB = 128


def _knn_mlp_body(xb_ref, ptc_ref, w1_ref, b1_ref, w2_ref, b2_ref,
                  idx_ref, h_ref):
    i = pl.program_id(1)
    n = xb_ref.shape[1] * pl.num_programs(1)
    nch = ptc_ref.shape[1]

    xb = xb_ref[0]
    h1 = jnp.maximum(
        jnp.dot(xb, w1_ref[...], preferred_element_type=jnp.float32)
        + b1_ref[...], 0.0)
    h_ref[0] = (jnp.dot(h1, w2_ref[...], preferred_element_type=jnp.float32)
                + b2_ref[...])

    lane8 = lax.broadcasted_iota(jnp.int32, (RB, 8), 1)
    xp = jnp.where(lane8 < 3, xb, 0.0)
    sqr = jnp.sum(xp * xp, axis=1)
    row_g = i * RB + lax.broadcasted_iota(jnp.int32, (RB, CW), 0)
    col_iota = lax.broadcasted_iota(jnp.int32, (RB, CW), 1)
    klane = lax.broadcasted_iota(jnp.int32, (RB, KNN), 1)
    lane2k = lax.broadcasted_iota(jnp.int32, (RB, 2 * KNN), 1)
    big = jnp.int32(2 ** 30)

    def chunk_step(c, carry):
        rd, ri = carry
        pt = ptc_ref[0, c]
        sqc = jnp.sum(pt * pt, axis=0)
        cross = jnp.dot(xp, pt, preferred_element_type=jnp.float32)
        d2 = sqr[:, None] + sqc[None, :] - 2.0 * cross
        col_g = c * CW + col_iota
        d2 = jnp.where(col_g == row_g, d2 + 1e10, d2)

        def extract(t, ec):
            d2c, cd, ci = ec
            m = jnp.min(d2c, axis=1)
            am = jnp.min(jnp.where(d2c <= m[:, None], col_g, big), axis=1)
            cd = jnp.where(klane == t, m[:, None], cd)
            ci = jnp.where(klane == t, am[:, None], ci)
            d2c = jnp.where(col_g == am[:, None], jnp.inf, d2c)
            return d2c, cd, ci

        cd0 = jnp.full((RB, KNN), jnp.inf, jnp.float32)
        ci0 = jnp.zeros((RB, KNN), jnp.int32)
        _, cd, ci = lax.fori_loop(0, KNN, extract, (d2, cd0, ci0))

        md = jnp.concatenate([rd, cd], axis=1)
        mi = jnp.concatenate([ri, ci], axis=1)

        def merge(t, mc):
            md, mi, nd, ni = mc
            m = jnp.min(md, axis=1)
            sel = md <= m[:, None]
            am = jnp.min(jnp.where(sel, mi, big), axis=1)
            nd = jnp.where(klane == t, m[:, None], nd)
            ni = jnp.where(klane == t, am[:, None], ni)
            md = jnp.where(sel & (mi == am[:, None]), jnp.inf, md)
            return md, mi, nd, ni

        _, _, nd, ni = lax.fori_loop(
            0, KNN, merge,
            (md, mi, jnp.zeros((RB, KNN), jnp.float32),
             jnp.zeros((RB, KNN), jnp.int32)))
        return nd, ni

    rd0 = jnp.full((RB, KNN), jnp.inf, jnp.float32)
    ri0 = jnp.zeros((RB, KNN), jnp.int32)
    _, ri = lax.fori_loop(0, nch, chunk_step, (rd0, ri0))
    idx_ref[0] = ri


def _knn_mlp(xb8, ptc, w1, b1, w2, b2):
    b, n, _ = xb8.shape
    nblk = n // RB
    grid = (b, nblk)
    return pl.pallas_call(
        _knn_mlp_body,
        grid=grid,
        in_specs=[
            pl.BlockSpec((1, RB, 8), lambda bi, i: (bi, i, 0)),
            pl.BlockSpec((1,) + ptc.shape[1:], lambda bi, i: (bi, 0, 0, 0)),
            pl.BlockSpec(w1.shape, lambda bi, i: (0, 0)),
            pl.BlockSpec(b1.shape, lambda bi, i: (0, 0)),
            pl.BlockSpec(w2.shape, lambda bi, i: (0, 0)),
            pl.BlockSpec(b2.shape, lambda bi, i: (0, 0)),
        ],
        out_specs=[
            pl.BlockSpec((1, RB, KNN), lambda bi, i: (bi, i, 0)),
            pl.BlockSpec((1, RB, HID), lambda bi, i: (bi, i, 0)),
        ],
        out_shape=[
            jax.ShapeDtypeStruct((b, n, KNN), jnp.int32),
            jax.ShapeDtypeStruct((b, n, HID), jnp.float32),
        ],
    )(xb8, ptc, w1, b1, w2, b2)


def _sc_gather(table, idx, chunk):
    info = plsc.get_sparse_core_info()
    nc, ns = info.num_cores, info.num_subcores
    nw = nc * ns
    e = idx.shape[0]
    d = table.shape[1]
    b_per_w = e // nw
    steps = b_per_w // chunk
    mesh = plsc.VectorSubcoreMesh(core_axis_name="c", subcore_axis_name="s")

    @functools.partial(
        pl.kernel, mesh=mesh,
        out_type=jax.ShapeDtypeStruct((e, d), jnp.float32),
        scratch_types=[
            pltpu.VMEM((chunk,), jnp.int32),
            pltpu.VMEM((chunk, d), jnp.float32),
            pltpu.SemaphoreType.DMA,
        ],
    )
    def gather_k(table_hbm, idx_hbm, out_hbm, idx_v, rows_v, sem):
        wid = lax.axis_index("s") * nc + lax.axis_index("c")
        base = wid * b_per_w

        def step(i, carry):
            off = base + i * chunk
            pltpu.sync_copy(idx_hbm.at[pl.ds(off, chunk)], idx_v)
            pltpu.async_copy(table_hbm.at[idx_v], rows_v, sem).wait()
            pltpu.sync_copy(rows_v, out_hbm.at[pl.ds(off, chunk)])
            return carry

        lax.fori_loop(0, steps, step, 0)

    return gather_k(table, idx)


def _pt_layer_body(g_ref, pg_ref, pi_ref, ws_ref, wv_ref, wp_ref, bp_ref,
                   out_ref, *, cout, reduce_max):
    g = g_ref[0].reshape(NBR * PB, HID)
    s = jnp.dot(g, ws_ref[...], preferred_element_type=jnp.float32)
    v = jnp.dot(g, wv_ref[...], preferred_element_type=jnp.float32)
    pg = pg_ref[0].reshape(NBR * PB, HID)[:, :16]
    pi = pi_ref[0]
    rel = jnp.broadcast_to(pi[None, :, :], (NBR, PB, 16)).reshape(
        NBR * PB, 16) - pg
    delta = (jnp.dot(rel, wp_ref[...], preferred_element_type=jnp.float32)
             + bp_ref[...])
    a = (delta - s).reshape(NBR, PB, cout)
    m = jnp.max(a, axis=0, keepdims=True)
    ex = jnp.exp(a - m)
    z = jnp.sum(ex, axis=0)
    w = ex * (v + delta).reshape(NBR, PB, cout)
    o = jnp.sum(w, axis=0) / z
    if reduce_max:
        out_ref[0, 0, 0] = jnp.max(o, axis=0)
    else:
        out_ref[0] = o


def _pt_layer(gt, posg, pos16, ws, wv, wp, bp, cout, reduce_max):
    b, _, n, _ = gt.shape
    nblk = n // PB
    body = functools.partial(_pt_layer_body, cout=cout, reduce_max=reduce_max)
    if reduce_max:
        out_spec = pl.BlockSpec((1, 1, 1, cout), lambda bi, i: (bi, i, 0, 0))
        out_shape = jax.ShapeDtypeStruct((b, nblk, 1, cout), jnp.float32)
    else:
        out_spec = pl.BlockSpec((1, PB, cout), lambda bi, i: (bi, i, 0))
        out_shape = jax.ShapeDtypeStruct((b, n, cout), jnp.float32)
    return pl.pallas_call(
        body,
        grid=(b, nblk),
        in_specs=[
            pl.BlockSpec((1, NBR, PB, HID), lambda bi, i: (bi, 0, i, 0)),
            pl.BlockSpec((1, NBR, PB, HID), lambda bi, i: (bi, 0, i, 0)),
            pl.BlockSpec((1, PB, 16), lambda bi, i: (bi, i, 0)),
            pl.BlockSpec(ws.shape, lambda bi, i: (0, 0)),
            pl.BlockSpec(wv.shape, lambda bi, i: (0, 0)),
            pl.BlockSpec(wp.shape, lambda bi, i: (0, 0)),
            pl.BlockSpec(bp.shape, lambda bi, i: (0, 0)),
        ],
        out_specs=out_spec,
        out_shape=out_shape,
    )(gt, posg, pos16, ws, wv, wp, bp)


def _pad_w(w, rows):
    return jnp.pad(w, ((0, rows - w.shape[0]), (0, 0)))


def kernel(x, params):
    b, cin, n = x.shape
    xb = jnp.transpose(x, (0, 2, 1))
    xb8 = jnp.pad(xb, ((0, 0), (0, 0), (0, 8 - cin)))
    post = jnp.pad(x[:, :3, :], ((0, 0), (0, 5), (0, 0)))
    nch = n // CW
    ptc = post.reshape(b, 8, nch, CW).transpose(0, 2, 1, 3)

    w1 = _pad_w(params['W1'], 8)
    b1 = params['b1'].reshape(1, HID)
    w2 = params['W2']
    b2 = params['b2'].reshape(1, HID)

    idx, h = _knn_mlp(xb8, ptc, w1, b1, w2, b2)

    self_col = jnp.broadcast_to(
        jnp.arange(n, dtype=jnp.int32)[None, :, None], (b, n, 1))
    nbr = jnp.concatenate([idx, self_col], axis=2)
    offs = (jnp.arange(b, dtype=jnp.int32) * n)[:, None, None]
    flat_idx = jnp.transpose(nbr + offs, (0, 2, 1)).reshape(b * n * NBR)

    pos16 = jnp.pad(xb[:, :, :3], ((0, 0), (0, 0), (0, 13)))
    pos128 = jnp.pad(xb[:, :, :3], ((0, 0), (0, 0), (0, HID - 3)))
    posg = _sc_gather(pos128.reshape(b * n, HID), flat_idx, 272)
    posg = posg.reshape(b, NBR, n, HID)

    for name, cout, last in (('pt1', HID, False), ('pt2', HID, False),
                             ('pt3', LAT, True)):
        p = params[name]
        gt = _sc_gather(h.reshape(b * n, HID), flat_idx, 272)
        gt = gt.reshape(b, NBR, n, HID)
        h = _pt_layer(gt, posg, pos16, p['Ws'], p['Wv'],
                      _pad_w(p['Wp'], 16), p['bp'].reshape(1, cout),
                      cout, last)

    return jnp.max(h, axis=(1, 2))


# speedup vs baseline: 7.0283x; 1.6208x over previous
import functools

import jax
import jax.numpy as jnp
from jax import lax
from jax.experimental import pallas as pl
from jax.experimental.pallas import tpu as pltpu
from jax.experimental.pallas import tpu_sc as plsc

KNN = 16
NBR = KNN + 1
CIN = 6
HID = 128
LAT = 256

RB = 128
CW = 1024
PB = 128


def _knn_mlp_body(xb_ref, ptc_ref, w1_ref, b1_ref, w2_ref, b2_ref,
                  idx_ref, h_ref):
    i = pl.program_id(1)
    n = xb_ref.shape[1] * pl.num_programs(1)
    nch = ptc_ref.shape[1]

    xb = xb_ref[0]
    h1 = jnp.maximum(
        jnp.dot(xb, w1_ref[...], preferred_element_type=jnp.float32)
        + b1_ref[...], 0.0)
    h_ref[0] = (jnp.dot(h1, w2_ref[...], preferred_element_type=jnp.float32)
                + b2_ref[...])

    lane8 = lax.broadcasted_iota(jnp.int32, (RB, 8), 1)
    xp = jnp.where(lane8 < 3, xb, 0.0)
    sqr = jnp.sum(xp * xp, axis=1)
    row_g = i * RB + lax.broadcasted_iota(jnp.int32, (RB, CW), 0)
    col_iota = lax.broadcasted_iota(jnp.int32, (RB, CW), 1)
    klane = lax.broadcasted_iota(jnp.int32, (RB, KNN), 1)
    lane2k = lax.broadcasted_iota(jnp.int32, (RB, 2 * KNN), 1)
    big = jnp.int32(2 ** 30)

    def chunk_step(c, carry):
        rd, ri = carry
        pt = ptc_ref[0, c]
        sqc = jnp.sum(pt * pt, axis=0)
        cross = jnp.dot(xp, pt, preferred_element_type=jnp.float32)
        d2 = sqr[:, None] + sqc[None, :] - 2.0 * cross
        col_g = c * CW + col_iota
        d2 = jnp.where(col_g == row_g, d2 + 1e10, d2)

        def extract(t, ec):
            d2c, cd, ci = ec
            m = jnp.min(d2c, axis=1)
            am = jnp.min(jnp.where(d2c <= m[:, None], col_g, big), axis=1)
            cd = jnp.where(klane == t, m[:, None], cd)
            ci = jnp.where(klane == t, am[:, None], ci)
            d2c = jnp.where(col_g == am[:, None], jnp.inf, d2c)
            return d2c, cd, ci

        cd0 = jnp.full((RB, KNN), jnp.inf, jnp.float32)
        ci0 = jnp.zeros((RB, KNN), jnp.int32)
        _, cd, ci = lax.fori_loop(0, KNN, extract, (d2, cd0, ci0))

        md = jnp.concatenate([rd, cd], axis=1)
        mi = jnp.concatenate([ri, ci], axis=1)

        def merge(t, mc):
            md, mi, nd, ni = mc
            m = jnp.min(md, axis=1)
            sel = md <= m[:, None]
            am = jnp.min(jnp.where(sel, mi, big), axis=1)
            nd = jnp.where(klane == t, m[:, None], nd)
            ni = jnp.where(klane == t, am[:, None], ni)
            md = jnp.where(sel & (mi == am[:, None]), jnp.inf, md)
            return md, mi, nd, ni

        _, _, nd, ni = lax.fori_loop(
            0, KNN, merge,
            (md, mi, jnp.zeros((RB, KNN), jnp.float32),
             jnp.zeros((RB, KNN), jnp.int32)))
        return nd, ni

    rd0 = jnp.full((RB, KNN), jnp.inf, jnp.float32)
    ri0 = jnp.zeros((RB, KNN), jnp.int32)
    _, ri = lax.fori_loop(0, nch, chunk_step, (rd0, ri0))
    idx_ref[0] = ri


def _knn_mlp(xb8, ptc, w1, b1, w2, b2):
    b, n, _ = xb8.shape
    nblk = n // RB
    grid = (b, nblk)
    return pl.pallas_call(
        _knn_mlp_body,
        grid=grid,
        in_specs=[
            pl.BlockSpec((1, RB, 8), lambda bi, i: (bi, i, 0)),
            pl.BlockSpec((1,) + ptc.shape[1:], lambda bi, i: (bi, 0, 0, 0)),
            pl.BlockSpec(w1.shape, lambda bi, i: (0, 0)),
            pl.BlockSpec(b1.shape, lambda bi, i: (0, 0)),
            pl.BlockSpec(w2.shape, lambda bi, i: (0, 0)),
            pl.BlockSpec(b2.shape, lambda bi, i: (0, 0)),
        ],
        out_specs=[
            pl.BlockSpec((1, RB, KNN), lambda bi, i: (bi, i, 0)),
            pl.BlockSpec((1, RB, HID), lambda bi, i: (bi, i, 0)),
        ],
        out_shape=[
            jax.ShapeDtypeStruct((b, n, KNN), jnp.int32),
            jax.ShapeDtypeStruct((b, n, HID), jnp.float32),
        ],
    )(xb8, ptc, w1, b1, w2, b2)


def _sc_gather(table, idx, chunk):
    info = plsc.get_sparse_core_info()
    nc, ns = info.num_cores, info.num_subcores
    nw = nc * ns
    e = idx.shape[0]
    d = table.shape[1]
    b_per_w = e // nw
    steps = b_per_w // chunk
    mesh = plsc.VectorSubcoreMesh(core_axis_name="c", subcore_axis_name="s")

    @functools.partial(
        pl.kernel, mesh=mesh,
        out_type=jax.ShapeDtypeStruct((e, d), jnp.float32),
        scratch_types=[
            pltpu.VMEM((chunk,), jnp.int32),
            pltpu.VMEM((chunk, d), jnp.float32),
            pltpu.SemaphoreType.DMA,
        ],
    )
    def gather_k(table_hbm, idx_hbm, out_hbm, idx_v, rows_v, sem):
        wid = lax.axis_index("s") * nc + lax.axis_index("c")
        base = wid * b_per_w

        def step(i, carry):
            off = base + i * chunk
            pltpu.sync_copy(idx_hbm.at[pl.ds(off, chunk)], idx_v)
            pltpu.async_copy(table_hbm.at[idx_v], rows_v, sem).wait()
            pltpu.sync_copy(rows_v, out_hbm.at[pl.ds(off, chunk)])
            return carry

        lax.fori_loop(0, steps, step, 0)

    return gather_k(table, idx)


def _pt_layer_body(g_ref, pg_ref, pi_ref, ws_ref, wv_ref, wp_ref, bp_ref,
                   out_ref, *, cout, reduce_max):
    g = g_ref[0].reshape(NBR * PB, HID)
    s = jnp.dot(g, ws_ref[...], preferred_element_type=jnp.float32)
    v = jnp.dot(g, wv_ref[...], preferred_element_type=jnp.float32)
    pg = pg_ref[0].reshape(NBR * PB, HID)[:, :16]
    pi = pi_ref[0]
    rel = jnp.broadcast_to(pi[None, :, :], (NBR, PB, 16)).reshape(
        NBR * PB, 16) - pg
    delta = (jnp.dot(rel, wp_ref[...], preferred_element_type=jnp.float32)
             + bp_ref[...])
    a = (delta - s).reshape(NBR, PB, cout)
    m = jnp.max(a, axis=0, keepdims=True)
    ex = jnp.exp(a - m)
    z = jnp.sum(ex, axis=0)
    w = ex * (v + delta).reshape(NBR, PB, cout)
    o = jnp.sum(w, axis=0) / z
    if reduce_max:
        out_ref[0, 0, 0] = jnp.max(o, axis=0)
    else:
        out_ref[0] = o


def _pt_layer(gt, posg, pos16, ws, wv, wp, bp, cout, reduce_max):
    b, _, n, _ = gt.shape
    nblk = n // PB
    body = functools.partial(_pt_layer_body, cout=cout, reduce_max=reduce_max)
    if reduce_max:
        out_spec = pl.BlockSpec((1, 1, 1, cout), lambda bi, i: (bi, i, 0, 0))
        out_shape = jax.ShapeDtypeStruct((b, nblk, 1, cout), jnp.float32)
    else:
        out_spec = pl.BlockSpec((1, PB, cout), lambda bi, i: (bi, i, 0))
        out_shape = jax.ShapeDtypeStruct((b, n, cout), jnp.float32)
    return pl.pallas_call(
        body,
        grid=(b, nblk),
        in_specs=[
            pl.BlockSpec((1, NBR, PB, HID), lambda bi, i: (bi, 0, i, 0)),
            pl.BlockSpec((1, NBR, PB, HID), lambda bi, i: (bi, 0, i, 0)),
            pl.BlockSpec((1, PB, 16), lambda bi, i: (bi, i, 0)),
            pl.BlockSpec(ws.shape, lambda bi, i: (0, 0)),
            pl.BlockSpec(wv.shape, lambda bi, i: (0, 0)),
            pl.BlockSpec(wp.shape, lambda bi, i: (0, 0)),
            pl.BlockSpec(bp.shape, lambda bi, i: (0, 0)),
        ],
        out_specs=out_spec,
        out_shape=out_shape,
    )(gt, posg, pos16, ws, wv, wp, bp)


def _pad_w(w, rows):
    return jnp.pad(w, ((0, rows - w.shape[0]), (0, 0)))


def kernel(x, params):
    b, cin, n = x.shape
    xb = jnp.transpose(x, (0, 2, 1))
    xb8 = jnp.pad(xb, ((0, 0), (0, 0), (0, 8 - cin)))
    post = jnp.pad(x[:, :3, :], ((0, 0), (0, 5), (0, 0)))
    nch = n // CW
    ptc = post.reshape(b, 8, nch, CW).transpose(0, 2, 1, 3)

    w1 = _pad_w(params['W1'], 8)
    b1 = params['b1'].reshape(1, HID)
    w2 = params['W2']
    b2 = params['b2'].reshape(1, HID)

    idx, h = _knn_mlp(xb8, ptc, w1, b1, w2, b2)

    self_col = jnp.broadcast_to(
        jnp.arange(n, dtype=jnp.int32)[None, :, None], (b, n, 1))
    nbr = jnp.concatenate([idx, self_col], axis=2)
    offs = (jnp.arange(b, dtype=jnp.int32) * n)[:, None, None]
    flat_idx = jnp.transpose(nbr + offs, (0, 2, 1)).reshape(b * n * NBR)

    pos16 = jnp.pad(xb[:, :, :3], ((0, 0), (0, 0), (0, 13)))
    pos128 = jnp.pad(xb[:, :, :3], ((0, 0), (0, 0), (0, HID - 3)))
    posg = _sc_gather(pos128.reshape(b * n, HID), flat_idx, 272)
    posg = posg.reshape(b, NBR, n, HID)

    for name, cout, last in (('pt1', HID, False), ('pt2', HID, False),
                             ('pt3', LAT, True)):
        p = params[name]
        gt = _sc_gather(h.reshape(b * n, HID), flat_idx, 272)
        gt = gt.reshape(b, NBR, n, HID)
        h = _pt_layer(gt, posg, pos16, p['Ws'], p['Wv'],
                      _pad_w(p['Wp'], 16), p['bp'].reshape(1, cout),
                      cout, last)

    return jnp.max(h, axis=(1, 2))


# speedup vs baseline: 9.9762x; 1.4194x over previous
import functools

import jax
import jax.numpy as jnp
from jax import lax
from jax.experimental import pallas as pl
from jax.experimental.pallas import tpu as pltpu
from jax.experimental.pallas import tpu_sc as plsc

KNN = 16
NBR = KNN + 1
CIN = 6
HID = 128
LAT = 256

RB = 128
CW = 2048
PB = 128


def _knn_mlp_body(xb_ref, ptc_ref, w1_ref, b1_ref, w2_ref, b2_ref,
                  idx_ref, h_ref):
    i = pl.program_id(1)
    n = xb_ref.shape[1] * pl.num_programs(1)
    nch = ptc_ref.shape[1]

    xb = xb_ref[0]
    h1 = jnp.maximum(
        jnp.dot(xb, w1_ref[...], preferred_element_type=jnp.float32)
        + b1_ref[...], 0.0)
    h_ref[0] = (jnp.dot(h1, w2_ref[...], preferred_element_type=jnp.float32)
                + b2_ref[...])

    lane8 = lax.broadcasted_iota(jnp.int32, (RB, 8), 1)
    xp = jnp.where(lane8 < 3, xb, 0.0)
    sqr = jnp.sum(xp * xp, axis=1)
    row_g = i * RB + lax.broadcasted_iota(jnp.int32, (RB, CW), 0)
    col_iota = lax.broadcasted_iota(jnp.int32, (RB, CW), 1)
    klane = lax.broadcasted_iota(jnp.int32, (RB, KNN), 1)
    lane2k = lax.broadcasted_iota(jnp.int32, (RB, 2 * KNN), 1)
    big = jnp.int32(2 ** 30)

    def chunk_step(c, carry):
        rd, ri = carry
        pt = ptc_ref[0, c]
        sqc = jnp.sum(pt * pt, axis=0)
        cross = jnp.dot(xp, pt, preferred_element_type=jnp.float32)
        d2 = sqr[:, None] + sqc[None, :] - 2.0 * cross
        col_g = c * CW + col_iota
        d2 = jnp.where(col_g == row_g, d2 + 1e10, d2)

        def extract(t, ec):
            d2c, cd, ci = ec
            m = jnp.min(d2c, axis=1)
            am = jnp.min(jnp.where(d2c <= m[:, None], col_g, big), axis=1)
            cd = jnp.where(klane == t, m[:, None], cd)
            ci = jnp.where(klane == t, am[:, None], ci)
            d2c = jnp.where(col_g == am[:, None], jnp.inf, d2c)
            return d2c, cd, ci

        cd0 = jnp.full((RB, KNN), jnp.inf, jnp.float32)
        ci0 = jnp.zeros((RB, KNN), jnp.int32)
        _, cd, ci = lax.fori_loop(0, KNN, extract, (d2, cd0, ci0))

        md = jnp.concatenate([rd, cd], axis=1)
        mi = jnp.concatenate([ri, ci], axis=1)

        def merge(t, mc):
            md, mi, nd, ni = mc
            m = jnp.min(md, axis=1)
            sel = md <= m[:, None]
            am = jnp.min(jnp.where(sel, mi, big), axis=1)
            nd = jnp.where(klane == t, m[:, None], nd)
            ni = jnp.where(klane == t, am[:, None], ni)
            md = jnp.where(sel & (mi == am[:, None]), jnp.inf, md)
            return md, mi, nd, ni

        _, _, nd, ni = lax.fori_loop(
            0, KNN, merge,
            (md, mi, jnp.zeros((RB, KNN), jnp.float32),
             jnp.zeros((RB, KNN), jnp.int32)))
        return nd, ni

    rd0 = jnp.full((RB, KNN), jnp.inf, jnp.float32)
    ri0 = jnp.zeros((RB, KNN), jnp.int32)
    _, ri = lax.fori_loop(0, nch, chunk_step, (rd0, ri0))
    idx_ref[0] = ri


def _knn_mlp(xb8, ptc, w1, b1, w2, b2):
    b, n, _ = xb8.shape
    nblk = n // RB
    grid = (b, nblk)
    return pl.pallas_call(
        _knn_mlp_body,
        grid=grid,
        in_specs=[
            pl.BlockSpec((1, RB, 8), lambda bi, i: (bi, i, 0)),
            pl.BlockSpec((1,) + ptc.shape[1:], lambda bi, i: (bi, 0, 0, 0)),
            pl.BlockSpec(w1.shape, lambda bi, i: (0, 0)),
            pl.BlockSpec(b1.shape, lambda bi, i: (0, 0)),
            pl.BlockSpec(w2.shape, lambda bi, i: (0, 0)),
            pl.BlockSpec(b2.shape, lambda bi, i: (0, 0)),
        ],
        out_specs=[
            pl.BlockSpec((1, RB, KNN), lambda bi, i: (bi, i, 0)),
            pl.BlockSpec((1, RB, HID), lambda bi, i: (bi, i, 0)),
        ],
        out_shape=[
            jax.ShapeDtypeStruct((b, n, KNN), jnp.int32),
            jax.ShapeDtypeStruct((b, n, HID), jnp.float32),
        ],
    )(xb8, ptc, w1, b1, w2, b2)


def _sc_gather(table, idx, chunk):
    info = plsc.get_sparse_core_info()
    nc, ns = info.num_cores, info.num_subcores
    nw = nc * ns
    e = idx.shape[0]
    d = table.shape[1]
    b_per_w = e // nw
    steps = b_per_w // chunk
    mesh = plsc.VectorSubcoreMesh(core_axis_name="c", subcore_axis_name="s")

    @functools.partial(
        pl.kernel, mesh=mesh,
        out_type=jax.ShapeDtypeStruct((e, d), jnp.float32),
        scratch_types=[
            pltpu.VMEM((chunk,), jnp.int32),
            pltpu.VMEM((chunk, d), jnp.float32),
            pltpu.SemaphoreType.DMA,
        ],
    )
    def gather_k(table_hbm, idx_hbm, out_hbm, idx_v, rows_v, sem):
        wid = lax.axis_index("s") * nc + lax.axis_index("c")
        base = wid * b_per_w

        def step(i, carry):
            off = base + i * chunk
            pltpu.sync_copy(idx_hbm.at[pl.ds(off, chunk)], idx_v)
            pltpu.async_copy(table_hbm.at[idx_v], rows_v, sem).wait()
            pltpu.sync_copy(rows_v, out_hbm.at[pl.ds(off, chunk)])
            return carry

        lax.fori_loop(0, steps, step, 0)

    return gather_k(table, idx)


def _pt_layer_body(g_ref, pg_ref, pi_ref, ws_ref, wv_ref, wp_ref, bp_ref,
                   out_ref, *, cout, reduce_max):
    g = g_ref[0].reshape(NBR * PB, HID)
    s = jnp.dot(g, ws_ref[...], preferred_element_type=jnp.float32)
    v = jnp.dot(g, wv_ref[...], preferred_element_type=jnp.float32)
    pg = pg_ref[0].reshape(NBR * PB, HID)[:, :16]
    pi = pi_ref[0]
    rel = jnp.broadcast_to(pi[None, :, :], (NBR, PB, 16)).reshape(
        NBR * PB, 16) - pg
    delta = (jnp.dot(rel, wp_ref[...], preferred_element_type=jnp.float32)
             + bp_ref[...])
    a = (delta - s).reshape(NBR, PB, cout)
    m = jnp.max(a, axis=0, keepdims=True)
    ex = jnp.exp(a - m)
    z = jnp.sum(ex, axis=0)
    w = ex * (v + delta).reshape(NBR, PB, cout)
    o = jnp.sum(w, axis=0) / z
    if reduce_max:
        out_ref[0, 0, 0] = jnp.max(o, axis=0)
    else:
        out_ref[0] = o


def _pt_layer(gt, posg, pos16, ws, wv, wp, bp, cout, reduce_max):
    b, _, n, _ = gt.shape
    nblk = n // PB
    body = functools.partial(_pt_layer_body, cout=cout, reduce_max=reduce_max)
    if reduce_max:
        out_spec = pl.BlockSpec((1, 1, 1, cout), lambda bi, i: (bi, i, 0, 0))
        out_shape = jax.ShapeDtypeStruct((b, nblk, 1, cout), jnp.float32)
    else:
        out_spec = pl.BlockSpec((1, PB, cout), lambda bi, i: (bi, i, 0))
        out_shape = jax.ShapeDtypeStruct((b, n, cout), jnp.float32)
    return pl.pallas_call(
        body,
        grid=(b, nblk),
        in_specs=[
            pl.BlockSpec((1, NBR, PB, HID), lambda bi, i: (bi, 0, i, 0)),
            pl.BlockSpec((1, NBR, PB, HID), lambda bi, i: (bi, 0, i, 0)),
            pl.BlockSpec((1, PB, 16), lambda bi, i: (bi, i, 0)),
            pl.BlockSpec(ws.shape, lambda bi, i: (0, 0)),
            pl.BlockSpec(wv.shape, lambda bi, i: (0, 0)),
            pl.BlockSpec(wp.shape, lambda bi, i: (0, 0)),
            pl.BlockSpec(bp.shape, lambda bi, i: (0, 0)),
        ],
        out_specs=out_spec,
        out_shape=out_shape,
    )(gt, posg, pos16, ws, wv, wp, bp)


def _pad_w(w, rows):
    return jnp.pad(w, ((0, rows - w.shape[0]), (0, 0)))


def kernel(x, params):
    b, cin, n = x.shape
    xb = jnp.transpose(x, (0, 2, 1))
    xb8 = jnp.pad(xb, ((0, 0), (0, 0), (0, 8 - cin)))
    post = jnp.pad(x[:, :3, :], ((0, 0), (0, 5), (0, 0)))
    nch = n // CW
    ptc = post.reshape(b, 8, nch, CW).transpose(0, 2, 1, 3)

    w1 = _pad_w(params['W1'], 8)
    b1 = params['b1'].reshape(1, HID)
    w2 = params['W2']
    b2 = params['b2'].reshape(1, HID)

    idx, h = _knn_mlp(xb8, ptc, w1, b1, w2, b2)

    self_col = jnp.broadcast_to(
        jnp.arange(n, dtype=jnp.int32)[None, :, None], (b, n, 1))
    nbr = jnp.concatenate([idx, self_col], axis=2)
    offs = (jnp.arange(b, dtype=jnp.int32) * n)[:, None, None]
    flat_idx = jnp.transpose(nbr + offs, (0, 2, 1)).reshape(b * n * NBR)

    pos16 = jnp.pad(xb[:, :, :3], ((0, 0), (0, 0), (0, 13)))
    pos128 = jnp.pad(xb[:, :, :3], ((0, 0), (0, 0), (0, HID - 3)))
    posg = _sc_gather(pos128.reshape(b * n, HID), flat_idx, 272)
    posg = posg.reshape(b, NBR, n, HID)

    for name, cout, last in (('pt1', HID, False), ('pt2', HID, False),
                             ('pt3', LAT, True)):
        p = params[name]
        gt = _sc_gather(h.reshape(b * n, HID), flat_idx, 272)
        gt = gt.reshape(b, NBR, n, HID)
        h = _pt_layer(gt, posg, pos16, p['Ws'], p['Wv'],
                      _pad_w(p['Wp'], 16), p['bp'].reshape(1, cout),
                      cout, last)

    return jnp.max(h, axis=(1, 2))
